# Initial kernel scaffold; baseline (speedup 1.0000x reference)
import jax
import jax.numpy as jnp
from jax.experimental import pallas as pl


def kernel(x, edge_index, W, b):
    raise NotImplementedError("write your pallas kernel here")


# speedup vs baseline: 11.9832x; 11.9832x over previous
import functools

import jax
import jax.numpy as jnp
from jax import lax
from jax.experimental import pallas as pl
from jax.experimental.pallas import tpu as pltpu
from jax.experimental.pallas import tpu_sc as plsc

N = 10000
NPAD = 10240
E = 320000
D = 128
DC = D // 2
NC, NS = 2, 16
NW = NC * NS
CH = 125
RPT = NPAD // NS

EW_D = E // NW
NCH_D = EW_D // CH
EW_A = E // NS
NCH_A = EW_A // CH

_mesh = plsc.VectorSubcoreMesh(core_axis_name="c", subcore_axis_name="s")


@functools.partial(
    pl.kernel,
    out_type=jax.ShapeDtypeStruct((NC, 2, NPAD), jnp.float32),
    mesh=_mesh,
    scratch_types=[
        pltpu.VMEM((NCH_D, CH), jnp.int32),
        pltpu.VMEM((NCH_D, CH), jnp.int32),
        pltpu.VMEM((128,), jnp.float32),
        pltpu.VMEM((RPT,), jnp.float32),
        pltpu.VMEM_SHARED((NPAD,), jnp.float32),
        pltpu.VMEM_SHARED((NPAD,), jnp.float32),
    ],
)
def _deg_kernel(ei_hbm, out_hbm, src_v, dst_v, ones_v, zero_v, ho_sh, hi_sh):
    c = lax.axis_index("c")
    s = lax.axis_index("s")
    wid = s * NC + c
    pltpu.sync_copy(ei_hbm.at[0, wid], src_v)
    pltpu.sync_copy(ei_hbm.at[1, wid], dst_v)

    def fill1(i, carry):
        ones_v[pl.ds(i * 16, 16)] = jnp.full((16,), 1.0, jnp.float32)
        return carry

    lax.fori_loop(0, 128 // 16, fill1, 0)

    def fill0(i, carry):
        zero_v[pl.ds(i * 16, 16)] = jnp.zeros((16,), jnp.float32)
        return carry

    lax.fori_loop(0, RPT // 16, fill0, 0)

    pltpu.sync_copy(zero_v, ho_sh.at[pl.ds(s * RPT, RPT)])
    pltpu.sync_copy(zero_v, hi_sh.at[pl.ds(s * RPT, RPT)])
    plsc.subcore_barrier()

    ones_ch = ones_v.at[pl.ds(0, CH)]

    def body(k, carry):
        pltpu.sync_copy(ones_ch, ho_sh.at[src_v.at[k]], add=True)
        pltpu.sync_copy(ones_ch, hi_sh.at[dst_v.at[k]], add=True)
        return carry

    lax.fori_loop(0, NCH_D, body, 0)
    plsc.subcore_barrier()

    pltpu.sync_copy(ho_sh.at[pl.ds(s * RPT, RPT)], out_hbm.at[c, 0, pl.ds(s * RPT, RPT)])
    pltpu.sync_copy(hi_sh.at[pl.ds(s * RPT, RPT)], out_hbm.at[c, 1, pl.ds(s * RPT, RPT)])


def _norm_body(deg_ref, x_ref, fa_ref, fb_ref, ninb_ref):
    p = deg_ref[...]
    do = p[0, 0, :N] + p[1, 0, :N]
    di = p[0, 1, :N] + p[1, 1, :N]
    no = lax.rsqrt(jnp.maximum(do, 1.0))
    ni = lax.rsqrt(jnp.maximum(di, 1.0))
    feat = x_ref[...] * no[:, None]
    fa_ref[...] = feat[:, :DC]
    fb_ref[...] = feat[:, DC:]
    ninb_ref[...] = jnp.broadcast_to(ni[:, None], (N, D))


_norm_call = pl.pallas_call(
    _norm_body,
    out_shape=[
        jax.ShapeDtypeStruct((N, DC), jnp.float32),
        jax.ShapeDtypeStruct((N, DC), jnp.float32),
        jax.ShapeDtypeStruct((N, D), jnp.float32),
    ],
)


@functools.partial(
    pl.kernel,
    out_type=jax.ShapeDtypeStruct((NC, NPAD, DC), jnp.float32),
    mesh=_mesh,
    scratch_types=[
        pltpu.VMEM((NCH_A, CH), jnp.int32),
        pltpu.VMEM((NCH_A, CH), jnp.int32),
        pltpu.VMEM((2, CH, DC), jnp.float32),
        pltpu.VMEM((128, DC), jnp.float32),
        pltpu.VMEM_SHARED((NPAD, DC), jnp.float32),
        pltpu.SemaphoreType.DMA,
        pltpu.SemaphoreType.DMA,
    ],
    compiler_params=pltpu.CompilerParams(use_tc_tiling_on_sc=False),
)
def _agg_kernel(fa_hbm, fb_hbm, ei_hbm, out_hbm, src_v, dst_v, rows_v,
                zbuf_v, agg_sh, sem0, sem1):
    c = lax.axis_index("c")
    s = lax.axis_index("s")
    pltpu.sync_copy(ei_hbm.at[0, s], src_v)
    pltpu.sync_copy(ei_hbm.at[1, s], dst_v)

    def fz(i, carry):
        r = i // 4
        q = i % 4
        zbuf_v[r, pl.ds(q * 16, 16)] = jnp.zeros((16,), jnp.float32)
        return carry

    lax.fori_loop(0, 128 * 4, fz, 0)

    def zc(j, carry):
        pltpu.sync_copy(zbuf_v, agg_sh.at[pl.ds(s * RPT + j * 128, 128)])
        return carry

    lax.fori_loop(0, RPT // 128, zc, 0)
    plsc.subcore_barrier()

    sems = (sem0, sem1)

    def start_gather(k, b):
        @pl.when(c == 0)
        def _():
            pltpu.async_copy(fa_hbm.at[src_v.at[k]], rows_v.at[b], sems[b])

        @pl.when(c == 1)
        def _():
            pltpu.async_copy(fb_hbm.at[src_v.at[k]], rows_v.at[b], sems[b])

    def wait_gather(k, b):
        pltpu.make_async_copy(fa_hbm.at[src_v.at[k]], rows_v.at[b],
                              sems[b]).wait()

    start_gather(0, 0)

    def step(k2, carry):
        for b in range(2):
            k = k2 * 2 + b
            nb = (b + 1) % 2

            @pl.when(k + 1 < NCH_A)
            def _():
                start_gather(k + 1, nb)

            wait_gather(k, b)
            pltpu.sync_copy(rows_v.at[b], agg_sh.at[dst_v.at[k]], add=True)
        return carry

    lax.fori_loop(0, NCH_A // 2, step, 0)
    plsc.subcore_barrier()

    pltpu.sync_copy(agg_sh.at[pl.ds(s * RPT, RPT)],
                    out_hbm.at[c, pl.ds(s * RPT, RPT)])


def _out_body(pa_ref, ninb_ref, w_ref, b_ref, o_ref):
    agg = jnp.concatenate([pa_ref[0], pa_ref[1]], axis=-1)
    rst = agg * ninb_ref[...]
    o_ref[...] = (
        jnp.dot(rst, w_ref[...], preferred_element_type=jnp.float32)
        + b_ref[...]
    )


_BLK = 1000
_out_call = pl.pallas_call(
    _out_body,
    grid=(N // _BLK,),
    in_specs=[
        pl.BlockSpec((NC, _BLK, DC), lambda i: (0, i, 0)),
        pl.BlockSpec((_BLK, D), lambda i: (i, 0)),
        pl.BlockSpec((D, D), lambda i: (0, 0)),
        pl.BlockSpec((1, D), lambda i: (0, 0)),
    ],
    out_specs=pl.BlockSpec((_BLK, D), lambda i: (i, 0)),
    out_shape=jax.ShapeDtypeStruct((N, D), jnp.float32),
)


def kernel(x, edge_index, W, b):
    ei32 = edge_index.astype(jnp.int32)
    ei_d = ei32.reshape(2, NW, NCH_D, CH)
    ei_a = ei32.reshape(2, NS, NCH_A, CH)
    deg = _deg_kernel(ei_d)
    fa, fb, ninb = _norm_call(deg, x)
    pagg = _agg_kernel(fa, fb, ei_a)
    return _out_call(pagg, ninb, W, b.reshape(1, D))
